# baseline (speedup 1.0000x reference)
import functools

import jax
import jax.numpy as jnp
from jax import lax
from jax.experimental import pallas as pl
from jax.experimental.pallas import tpu as pltpu
from jax.experimental.pallas import tpu_sc as plsc

B = 16384
C = 128
LANES = 16
NCORES = 2
NSUBCORES = 16
NUM_WORKERS = NCORES * NSUBCORES
ROWS_PER_WORKER = B // NUM_WORKERS
CVECS = C // LANES
NEXPERTS = 64


def _sc_body(in_hbm, idx_hbm, w_hbm, fact_hbm, out_hbm,
             idx_v, tbl_f, fact_v, in_f,
             sem_stage, sem_in, sem_out):
    wid = lax.axis_index("s") * NCORES + lax.axis_index("c")
    row0 = wid * ROWS_PER_WORKER

    c_in = pltpu.async_copy(in_hbm.at[pl.ds(row0 * C, ROWS_PER_WORKER * C)],
                            in_f, sem_in)
    c_idx = pltpu.async_copy(idx_hbm.at[pl.ds(wid, 1), :], idx_v, sem_stage)
    c_tbl = pltpu.async_copy(w_hbm, tbl_f, sem_stage)
    c_fact = pltpu.async_copy(fact_hbm, fact_v, sem_stage)
    c_tbl.wait()
    c_fact.wait()

    fact_vs = [fact_v[pl.ds(j * LANES, LANES)] for j in range(CVECS)]

    @plsc.parallel_loop(0, NEXPERTS, step=1, unroll=4)
    def add_fact(e):
        eo = e * C
        for j in range(CVECS):
            sl = pl.ds(eo + j * LANES, LANES)
            tbl_f[sl] = tbl_f[sl] + fact_vs[j]

    c_idx.wait()
    c_in.wait()

    @plsc.parallel_loop(0, ROWS_PER_WORKER // LANES, step=1, unroll=2)
    def grp_body(rg):
        eo_v = idx_v[0, pl.ds(rg * LANES, LANES)] * C
        base = rg * (LANES * C)
        eos = [eo_v[t] for t in range(LANES)]
        for t in range(LANES):
            eo = eos[t]
            ro = base + t * C
            for j in range(CVECS):
                iv = in_f[pl.ds(ro + j * LANES, LANES)]
                sv = tbl_f[pl.ds(eo + j * LANES, LANES)]
                in_f[pl.ds(ro + j * LANES, LANES)] = jnp.where(
                    iv >= 0.0, iv, sv * iv)

    pltpu.async_copy(in_f, out_hbm.at[pl.ds(row0 * C, ROWS_PER_WORKER * C)],
                     sem_out).wait()


@jax.jit
def _run(input, route_index, weight, weight_fact):
    mesh = plsc.VectorSubcoreMesh(core_axis_name="c", subcore_axis_name="s")
    f = functools.partial(
        pl.kernel,
        out_type=jax.ShapeDtypeStruct((B * C,), jnp.float32),
        mesh=mesh,
        scratch_types=[
            pltpu.VMEM((1, ROWS_PER_WORKER), jnp.int32),
            pltpu.VMEM((NEXPERTS * C,), jnp.float32),
            pltpu.VMEM((C,), jnp.float32),
            pltpu.VMEM((ROWS_PER_WORKER * C,), jnp.float32),
            pltpu.SemaphoreType.DMA,
            pltpu.SemaphoreType.DMA,
            pltpu.SemaphoreType.DMA,
        ],
    )(_sc_body)
    idx2d = route_index.astype(jnp.int32).reshape(NUM_WORKERS, ROWS_PER_WORKER)
    out = f(input.reshape(B * C), idx2d, weight.reshape(NEXPERTS * C),
            weight_fact.reshape(C))
    return out.reshape(B, C)


def kernel(input, route_index, weight, weight_fact):
    return _run(input, route_index, weight, weight_fact)


# speedup vs baseline: 1.3341x; 1.3341x over previous
import functools

import jax
import jax.numpy as jnp
from jax import lax
from jax.experimental import pallas as pl
from jax.experimental.pallas import tpu as pltpu
from jax.experimental.pallas import tpu_sc as plsc

B = 16384
C = 128
LANES = 16
NCORES = 2
NSUBCORES = 16
NUM_WORKERS = NCORES * NSUBCORES
ROWS_PER_WORKER = B // NUM_WORKERS
CVECS = C // LANES
NEXPERTS = 64


def _sc_body(in_hbm, idx_hbm, w_hbm, fact_hbm, out_hbm,
             idx_v, tbl_f, fact_v, in_f,
             sem_stage, sem_in, sem_out):
    wid = lax.axis_index("s") * NCORES + lax.axis_index("c")
    row0 = wid * ROWS_PER_WORKER

    c_in = pltpu.async_copy(in_hbm.at[pl.ds(row0 * C, ROWS_PER_WORKER * C)],
                            in_f, sem_in)
    c_idx = pltpu.async_copy(idx_hbm.at[pl.ds(wid, 1), :], idx_v, sem_stage)
    c_tbl = pltpu.async_copy(w_hbm, tbl_f, sem_stage)
    c_fact = pltpu.async_copy(fact_hbm, fact_v, sem_stage)
    c_tbl.wait()
    c_fact.wait()

    fact_vs = [fact_v[pl.ds(j * LANES, LANES)] for j in range(CVECS)]

    @plsc.parallel_loop(0, NEXPERTS, step=1, unroll=4)
    def add_fact(e):
        eo = e * C
        for j in range(CVECS):
            sl = pl.ds(eo + j * LANES, LANES)
            tbl_f[sl] = tbl_f[sl] + fact_vs[j]

    c_idx.wait()
    c_in.wait()

    @plsc.parallel_loop(0, ROWS_PER_WORKER // LANES, step=1, unroll=1)
    def grp_body(rg):
        eo_v = idx_v[0, pl.ds(rg * LANES, LANES)] * C
        base = rg * (LANES * C)
        eos = [eo_v[t] for t in range(LANES)]
        for t in range(LANES):
            eo = eos[t]
            ro = base + t * C
            for j in range(CVECS):
                iv = in_f[pl.ds(ro + j * LANES, LANES)]
                sv = tbl_f[pl.ds(eo + j * LANES, LANES)]
                in_f[pl.ds(ro + j * LANES, LANES)] = jnp.where(
                    iv >= 0.0, iv, sv * iv)

    pltpu.async_copy(in_f, out_hbm.at[pl.ds(row0 * C, ROWS_PER_WORKER * C)],
                     sem_out).wait()


@jax.jit
def _run(input, route_index, weight, weight_fact):
    mesh = plsc.VectorSubcoreMesh(core_axis_name="c", subcore_axis_name="s")
    f = functools.partial(
        pl.kernel,
        out_type=jax.ShapeDtypeStruct((B * C,), jnp.float32),
        mesh=mesh,
        scratch_types=[
            pltpu.VMEM((1, ROWS_PER_WORKER), jnp.int32),
            pltpu.VMEM((NEXPERTS * C,), jnp.float32),
            pltpu.VMEM((C,), jnp.float32),
            pltpu.VMEM((ROWS_PER_WORKER * C,), jnp.float32),
            pltpu.SemaphoreType.DMA,
            pltpu.SemaphoreType.DMA,
            pltpu.SemaphoreType.DMA,
        ],
    )(_sc_body)
    idx2d = route_index.astype(jnp.int32).reshape(NUM_WORKERS, ROWS_PER_WORKER)
    out = f(input.reshape(B * C), idx2d, weight.reshape(NEXPERTS * C),
            weight_fact.reshape(C))
    return out.reshape(B, C)


def kernel(input, route_index, weight, weight_fact):
    return _run(input, route_index, weight, weight_fact)
